# baseline (speedup 1.0000x reference)
import functools

import jax
import jax.numpy as jnp
from jax import lax
from jax.experimental import pallas as pl
from jax.experimental.pallas import tpu as pltpu
from jax.experimental.pallas import tpu_sc as plsc

_B = 1024
_NEG = 200
_D = 128
_L = 16
_NC = 2
_NS = 16
_NW = _NC * _NS
_BPW = _B // _NW
_CA = 104
_CB = 96
_NROWS = 208
_DEPTH = 4


def _sc_body(heads, rels, tails, negs, e_tab, r_tab,
             dneg_out, dpos_out,
             pidx_h, pidx_r, pidx_t, u_rows, dpos_v, idx_all,
             nrows0, nrows1, nrows2, nrows3, dist0, dist1, dist2, dist3,
             sem_p, sem0, sem1, sem2, sem3, semw0, semw1, semw2, semw3):
    wid = lax.axis_index("s") * _NC + lax.axis_index("c")
    base = pl.multiple_of(wid * _BPW, _BPW)
    lanes = lax.iota(jnp.int32, _L)
    bufs = ((nrows0, dist0, sem0, semw0), (nrows1, dist1, sem1, semw1),
            (nrows2, dist2, sem2, semw2), (nrows3, dist3, sem3, semw3))

    pltpu.sync_copy(negs.at[pl.ds(pl.multiple_of(base * _NEG, 8), _BPW * _NEG)],
                    idx_all)

    pltpu.sync_copy(heads.at[pl.ds(base, _BPW)], pidx_h)
    pltpu.sync_copy(rels.at[pl.ds(base, _BPW)], pidx_r)
    pltpu.sync_copy(tails.at[pl.ds(base, _BPW)], pidx_t)
    pltpu.async_copy(e_tab.at[pidx_h], nrows3.at[pl.ds(0, _BPW)], sem_p)
    pltpu.async_copy(r_tab.at[pidx_r], nrows3.at[pl.ds(_BPW, _BPW)], sem_p)
    pltpu.async_copy(e_tab.at[pidx_t], nrows3.at[pl.ds(2 * _BPW, _BPW)], sem_p)

    def _issue(b_loc, nrows, sem):
        offa = pl.multiple_of(b_loc * _NEG, 8)
        offb = pl.multiple_of(b_loc * _NEG + _CA, 8)
        pltpu.async_copy(e_tab.at[idx_all.at[pl.ds(offa, _CA)]],
                         nrows.at[pl.ds(0, _CA)], sem)
        pltpu.async_copy(e_tab.at[idx_all.at[pl.ds(offb, _CB)]],
                         nrows.at[pl.ds(_CA, _CB)], sem)

    def _drain(b_loc, nrows, sem):
        offa = pl.multiple_of(b_loc * _NEG, 8)
        offb = pl.multiple_of(b_loc * _NEG + _CA, 8)
        pltpu.make_async_copy(e_tab.at[idx_all.at[pl.ds(offa, _CA)]],
                              nrows.at[pl.ds(0, _CA)], sem).wait()
        pltpu.make_async_copy(e_tab.at[idx_all.at[pl.ds(offb, _CB)]],
                              nrows.at[pl.ds(_CA, _CB)], sem).wait()

    _issue(0, nrows0, sem0)
    _issue(1, nrows1, sem1)
    _issue(2, nrows2, sem2)

    pltpu.make_async_copy(e_tab.at[pidx_h], nrows3.at[pl.ds(0, _BPW)],
                          sem_p).wait()
    pltpu.make_async_copy(r_tab.at[pidx_r], nrows3.at[pl.ds(_BPW, _BPW)],
                          sem_p).wait()
    pltpu.make_async_copy(e_tab.at[pidx_t], nrows3.at[pl.ds(2 * _BPW, _BPW)],
                          sem_p).wait()

    @pl.loop(0, _BPW)
    def _(b):
        for c in range(_D // _L):
            sl = pl.ds(c * _L, _L)
            u_rows[b, sl] = nrows3[b, sl] + nrows3[_BPW + b, sl]

    zero_v = jnp.zeros((_L,), jnp.float32)

    def _tree_add(vs):
        while len(vs) > 1:
            vs = [a + b for a, b in zip(vs[::2], vs[1::2])]
        return vs[0]

    def _lane_sum(v):
        for sh in (8, 4, 2, 1):
            perm = jnp.bitwise_xor(lanes, sh)
            v = v + jnp.take_along_axis(v, perm, axis=0,
                                        mode="promise_in_bounds")
        return v

    def _l1_row(rows, r, u_vecs):
        diffs = [jnp.abs(u_vecs[c] - rows[r, pl.ds(c * _L, _L)])
                 for c in range(_D // _L)]
        return _lane_sum(_tree_add(diffs))

    for rb in range(_BPW // _L):
        def _pos_j(j, dvec, rb=rb):
            b = rb * _L + j
            u_vecs = [u_rows[b, pl.ds(c * _L, _L)] for c in range(_D // _L)]
            sv = _l1_row(nrows3, 2 * _BPW + b, u_vecs)
            return jnp.where(lanes == j, sv, dvec)

        dvec = lax.fori_loop(0, _L, _pos_j, zero_v, unroll=True)
        dpos_v[pl.ds(rb * _L, _L)] = dvec
    pltpu.sync_copy(dpos_v, dpos_out.at[pl.ds(base, _BPW)])

    _issue(3, nrows3, sem3)

    def _compute(b_loc, nrows, dist):
        u_vecs = [u_rows[b_loc, pl.ds(c * _L, _L)] for c in range(_D // _L)]

        @pl.loop(0, _NROWS // _L)
        def _(rb):
            def _neg_j(j, dvec):
                sv = _l1_row(nrows, rb * _L + j, u_vecs)
                return jnp.where(lanes == j, sv, dvec)

            dvec = lax.fori_loop(0, _L, _neg_j, zero_v, unroll=True)
            dist[pl.ds(pl.multiple_of(rb * _L, _L), _L)] = dvec

    def _dist_write(b_loc, dist, semw):
        off = pl.multiple_of((base + b_loc) * _NEG, 8)
        pltpu.async_copy(dist.at[pl.ds(0, _NEG)], dneg_out.at[pl.ds(off, _NEG)],
                         semw)

    def _dist_drain(b_loc, dist, semw):
        off = pl.multiple_of((base + b_loc) * _NEG, 8)
        pltpu.make_async_copy(dist.at[pl.ds(0, _NEG)],
                              dneg_out.at[pl.ds(off, _NEG)], semw).wait()

    @pl.loop(0, _BPW // _DEPTH)
    def _(g):
        for buf, (nrows, dist, sem, semw) in enumerate(bufs):
            b = g * _DEPTH + buf
            _drain(b, nrows, sem)

            @pl.when(b >= _DEPTH)
            def _():
                _dist_drain(b - _DEPTH, dist, semw)

            _compute(b, nrows, dist)
            _dist_write(b, dist, semw)

            @pl.when(b + _DEPTH < _BPW)
            def _():
                _issue(b + _DEPTH, nrows, sem)

    for buf, (nrows, dist, sem, semw) in enumerate(bufs):
        _dist_drain(_BPW - _DEPTH + buf, dist, semw)


_sc_distances = functools.partial(
    pl.kernel,
    out_type=[
        jax.ShapeDtypeStruct((_B * _NEG,), jnp.float32),
        jax.ShapeDtypeStruct((_B,), jnp.float32),
    ],
    mesh=plsc.VectorSubcoreMesh(core_axis_name="c", subcore_axis_name="s"),
    compiler_params=pltpu.CompilerParams(needs_layout_passes=False),
    scratch_types=[
        pltpu.VMEM((_BPW,), jnp.int32),
        pltpu.VMEM((_BPW,), jnp.int32),
        pltpu.VMEM((_BPW,), jnp.int32),
        pltpu.VMEM((_BPW, _D), jnp.float32),
        pltpu.VMEM((_BPW,), jnp.float32),
        pltpu.VMEM((_BPW * _NEG,), jnp.int32),
        pltpu.VMEM((_NROWS, _D), jnp.float32),
        pltpu.VMEM((_NROWS, _D), jnp.float32),
        pltpu.VMEM((_NROWS, _D), jnp.float32),
        pltpu.VMEM((_NROWS, _D), jnp.float32),
        pltpu.VMEM((_NROWS,), jnp.float32),
        pltpu.VMEM((_NROWS,), jnp.float32),
        pltpu.VMEM((_NROWS,), jnp.float32),
        pltpu.VMEM((_NROWS,), jnp.float32),
        pltpu.SemaphoreType.DMA,
        pltpu.SemaphoreType.DMA,
        pltpu.SemaphoreType.DMA,
        pltpu.SemaphoreType.DMA,
        pltpu.SemaphoreType.DMA,
        pltpu.SemaphoreType.DMA,
        pltpu.SemaphoreType.DMA,
        pltpu.SemaphoreType.DMA,
        pltpu.SemaphoreType.DMA,
    ],
)(_sc_body)


def _tc_body(dneg_ref, dpos_ref, out_ref):
    s = dneg_ref[...]
    neg_loss = jnp.sum(jnp.log1p(jnp.exp(-s))) / (_B * _NEG)
    p = dpos_ref[...]
    pos_loss = jnp.sum(p + jnp.log1p(jnp.exp(-p))) / _B
    out_ref[...] = jnp.reshape(0.5 * (pos_loss + neg_loss), (1, 1))


_tc_loss = pl.pallas_call(
    _tc_body,
    out_shape=jax.ShapeDtypeStruct((1, 1), jnp.float32),
)


def kernel(positive_sample, negative_sample, subsample_weight, E_emb, R_emb):
    heads = positive_sample[:, 0].astype(jnp.int32)
    rels = positive_sample[:, 1].astype(jnp.int32)
    tails = positive_sample[:, 2].astype(jnp.int32)
    negs = negative_sample.reshape(-1).astype(jnp.int32)
    dneg, dpos = _sc_distances(heads, rels, tails, negs,
                               E_emb.astype(jnp.float32),
                               R_emb.astype(jnp.float32))
    loss = _tc_loss(dneg.reshape(_B, _NEG), dpos.reshape(8, _D))
    return loss[0, 0]
